# baseline (speedup 1.0000x reference)
import functools

import jax
import jax.numpy as jnp
from jax import lax
from jax.experimental import pallas as pl
from jax.experimental.pallas import tpu as pltpu
from jax.experimental.pallas import tpu_sc as plsc

N = 10000
NPAD = 10240
EP = 320000
EL = 160000
HID = 128
WP = 144
NC, NS = 2, 16
NW = NC * NS
CH = 128
G = 2
CP0, CP1 = 112, 48
CL0, CL1 = 56, 24
CP0C, CP1C = 112, 48
CL0C, CL1C = 56, 24
EPP = (CP0 + CP1) * NS * CH // 2 * 2
ELP = (CL0 + CL1) * NS * CH // 2 * 2
EPP = 2560 * CH
ELP = 1280 * CH
ROWS_PER_TILE = NPAD // NS

_mesh = plsc.VectorSubcoreMesh(core_axis_name="c", subcore_axis_name="s")


def _stream_pipe(table, acc, comb, base_row, ngroups, idxbuf, rowbuf,
                 gsem, isem, goff, soff):
    RG = 2 * G

    def issue(bank, r, p):
        pltpu.async_copy(table.at[idxbuf.at[bank, 2 * r + goff]],
                         rowbuf.at[p], gsem)

    def wait_gather(bank, r, p):
        pltpu.make_async_copy(table.at[idxbuf.at[bank, 2 * r + goff]],
                              rowbuf.at[p], gsem).wait()

    pltpu.sync_copy(comb.at[pl.ds(base_row, RG)], idxbuf.at[0])
    issue(0, 0, 0)

    @pl.when(ngroups > 1)
    def _():
        pltpu.async_copy(comb.at[pl.ds(base_row + RG, RG)], idxbuf.at[1],
                         isem)

    def group_body(g, _):
        b = lax.rem(g, 2)

        def chunk_body(r, _):
            k = g * G + r
            p = lax.rem(k, 2)
            wait_gather(b, r, p)

            @pl.when(r < G - 1)
            def _():
                issue(b, r + 1, 1 - p)

            pltpu.sync_copy(rowbuf.at[p], acc.at[idxbuf.at[b, 2 * r + soff]],
                            add=True)
            return 0
        lax.fori_loop(0, G, chunk_body, 0)

        @pl.when(g < ngroups - 1)
        def _():
            pltpu.make_async_copy(
                comb.at[pl.ds(base_row + (g + 1) * RG, RG)],
                idxbuf.at[1 - b], isem).wait()
            issue(1 - b, 0, 0)

            @pl.when(g < ngroups - 2)
            def __():
                pltpu.async_copy(
                    comb.at[pl.ds(base_row + (g + 2) * RG, RG)],
                    idxbuf.at[b], isem)
        return 0
    lax.fori_loop(0, ngroups, group_body, 0)


@functools.partial(
    pl.kernel,
    out_type=jax.ShapeDtypeStruct((NC, NPAD, WP), jnp.float32),
    mesh=_mesh,
    compiler_params=pltpu.CompilerParams(use_tc_tiling_on_sc=False),
    scratch_types=[
        pltpu.VMEM((2, 2 * G, CH), jnp.int32),
        pltpu.VMEM((2, CH, WP), jnp.float32),
        pltpu.VMEM_SHARED((NPAD, WP), jnp.float32),
        pltpu.SemaphoreType.DMA,
        pltpu.SemaphoreType.DMA,
    ],
)
def _sc_stage_a1(xp144, combp, zrow144, aggp_out, idxbuf, rowbuf, accp,
                 gsem, isem):
    cid = lax.axis_index("c")
    sid = lax.axis_index("s")
    sl = pl.ds(sid * ROWS_PER_TILE, ROWS_PER_TILE)

    pltpu.sync_copy(zrow144, accp.at[sl])
    plsc.subcore_barrier()

    cntp = CP0 + cid * (CP1 - CP0)
    basep = cid * (NS * CP0) + sid * cntp
    _stream_pipe(xp144, accp, combp, basep * 2, cntp // G, idxbuf, rowbuf,
                 gsem, isem, goff=0, soff=1)

    plsc.subcore_barrier()
    pltpu.sync_copy(accp.at[sl], aggp_out.at[cid, sl])


@functools.partial(
    pl.kernel,
    out_type=jax.ShapeDtypeStruct((NC, NPAD, 16), jnp.float32),
    mesh=_mesh,
    compiler_params=pltpu.CompilerParams(use_tc_tiling_on_sc=False),
    scratch_types=[
        pltpu.VMEM((2, 2 * G, CH), jnp.int32),
        pltpu.VMEM((2, CH, 16), jnp.float32),
        pltpu.VMEM_SHARED((NPAD, 16), jnp.float32),
        pltpu.SemaphoreType.DMA,
        pltpu.SemaphoreType.DMA,
    ],
)
def _sc_stage_a2(xl16, combl, zrow16, aggl_out, idxbuf, rowbuf16, accl,
                 gsem, isem):
    cid = lax.axis_index("c")
    sid = lax.axis_index("s")
    sl = pl.ds(sid * ROWS_PER_TILE, ROWS_PER_TILE)

    pltpu.sync_copy(zrow16, accl.at[sl])
    plsc.subcore_barrier()

    cntl = CL0 + cid * (CL1 - CL0)
    basel = cid * (NS * CL0) + sid * cntl
    _stream_pipe(xl16, accl, combl, basel * 2, cntl // G, idxbuf, rowbuf16,
                 gsem, isem, goff=0, soff=1)

    plsc.subcore_barrier()
    pltpu.sync_copy(accl.at[sl], aggl_out.at[cid, sl])


@functools.partial(
    pl.kernel,
    out_type=[
        jax.ShapeDtypeStruct((NC, NPAD, HID), jnp.float32),
        jax.ShapeDtypeStruct((NC, NPAD, 16), jnp.float32),
    ],
    mesh=_mesh,
    compiler_params=pltpu.CompilerParams(use_tc_tiling_on_sc=False),
    scratch_types=[
        pltpu.VMEM((2, 2 * G, CH), jnp.int32),
        pltpu.VMEM((2, CH, HID), jnp.float32),
        pltpu.VMEM((2, CH, 16), jnp.float32),
        pltpu.VMEM_SHARED((NPAD, HID), jnp.float32),
        pltpu.VMEM_SHARED((NPAD, 16), jnp.float32),
        pltpu.SemaphoreType.DMA,
        pltpu.SemaphoreType.DMA,
    ],
)
def _sc_stage_c(z1l, invp16, combp, combl, zrow128, zrow16,
                aggl2_out, w_out,
                idxbuf, rowbuf, rowbuf16, accl2, wacc, gsem, isem):
    cid = lax.axis_index("c")
    sid = lax.axis_index("s")
    sl = pl.ds(sid * ROWS_PER_TILE, ROWS_PER_TILE)

    pltpu.sync_copy(zrow128, accl2.at[sl])
    pltpu.sync_copy(zrow16, wacc.at[sl])
    plsc.subcore_barrier()

    cntp = CP0C + cid * (CP1C - CP0C)
    basep = cid * (NS * CP0C) + sid * cntp
    cntl = CL0C + cid * (CL1C - CL0C)
    basel = cid * (NS * CL0C) + sid * cntl
    _stream_pipe(z1l, accl2, combl, basel * 2, cntl // G, idxbuf, rowbuf,
                 gsem, isem, goff=0, soff=1)
    _stream_pipe(invp16, wacc, combp, basep * 2, cntp // G, idxbuf, rowbuf16,
                 gsem, isem, goff=1, soff=0)

    plsc.subcore_barrier()
    pltpu.sync_copy(accl2.at[sl], aggl2_out.at[cid, sl])
    pltpu.sync_copy(wacc.at[sl], w_out.at[cid, sl])


BR = 1280
GRID = NPAD // BR


def _tc_stage_b(xp_ref, aggp0_ref, aggp1_ref, degp0_ref, degp1_ref,
                xl16_ref, aggl0_ref, aggl1_ref, wp1_ref, wl1p_ref,
                hp_ref, z1l_ref, invp16_ref, invl16_ref):
    i = pl.program_id(0)
    rows = lax.broadcasted_iota(jnp.int32, (BR, 1), 0) + i * BR
    mask = rows < N

    degp = degp0_ref[:, :1] + degp1_ref[:, :1]
    invp = jnp.where(mask, 1.0 / jnp.maximum(degp, 1.0), 0.0)
    aggp = aggp0_ref[...] + aggp1_ref[...]
    hp = jnp.maximum(jnp.dot(aggp * invp + xp_ref[...], wp1_ref[...],
                             preferred_element_type=jnp.float32), 0.0)
    hp_ref[...] = jnp.where(mask, hp, 0.0)

    aggl = aggl0_ref[...] + aggl1_ref[...]
    degl = aggl[:, 15:16]
    invl = jnp.where(mask, 1.0 / jnp.maximum(degl, 1.0), 0.0)
    z1 = jnp.maximum(jnp.dot(aggl * invl + xl16_ref[...], wl1p_ref[...],
                             preferred_element_type=jnp.float32), 0.0)
    z1l_ref[...] = jnp.where(mask, z1, 0.0)

    invp16_ref[...] = jnp.broadcast_to(invp, (BR, 16))
    invl16_ref[...] = jnp.broadcast_to(invl, (BR, 16))


def _tc_stage_d(hp_ref, z1l_ref, aggl20_ref, aggl21_ref, invl16_ref,
                lab16_ref, w0_ref, w1_ref, wl2_ref, wf16_ref, bf16_ref,
                wg1p_ref, sums_ref):
    i = pl.program_id(0)
    rows = lax.broadcasted_iota(jnp.int32, (BR, 1), 0) + i * BR
    mask = rows < N

    invl = invl16_ref[:, :1]
    zv = jnp.dot((aggl20_ref[...] + aggl21_ref[...]) * invl + z1l_ref[...],
                 wl2_ref[...], preferred_element_type=jnp.float32)

    lab = lab16_ref[...]
    logits = jnp.dot(zv, wf16_ref[...], preferred_element_type=jnp.float32) \
        + bf16_ref[...]
    lane = lax.broadcasted_iota(jnp.int32, (BR, 16), 1)
    lmask = lane < 10
    m = jnp.max(jnp.where(lmask, logits, -3e38), axis=1, keepdims=True)
    p = jnp.where(lmask, jnp.exp(logits - m), 0.0)
    val = jnp.sum(p * lab, axis=1, keepdims=True) / jnp.sum(p, axis=1, keepdims=True)
    logterm = jnp.where(mask, jnp.log(val + 1e-12), 0.0)

    hp = hp_ref[...]
    wrow = w0_ref[:, :1] + w1_ref[:, :1]
    relu_g = jnp.maximum(jnp.dot(lab, wg1p_ref[...],
                                 preferred_element_type=jnp.float32), 0.0)

    r_zv = jnp.sum(zv, axis=0, keepdims=True)
    r_hp = jnp.sum(hp, axis=0, keepdims=True)
    r_wh = jnp.sum(wrow * hp, axis=0, keepdims=True)
    r_rg = jnp.sum(relu_g, axis=0, keepdims=True)
    r_lab = jnp.concatenate(
        [jnp.sum(lab, axis=0, keepdims=True), jnp.zeros((1, 112), jnp.float32)],
        axis=1)
    lane128 = lax.broadcasted_iota(jnp.int32, (1, 128), 1)
    r_log = jnp.where(lane128 == 0, jnp.sum(logterm), 0.0)
    add = jnp.concatenate(
        [r_zv, r_hp, r_wh, r_rg, r_lab, r_log, jnp.zeros((2, 128), jnp.float32)],
        axis=0)

    @pl.when(i == 0)
    def _():
        sums_ref[...] = jnp.zeros((8, 128), jnp.float32)

    sums_ref[...] += add


def _tc_stage_d2(sums_ref, lab16_ref, wp2_ref, wg2_ref, wg1p_ref, bfs_ref,
                 out_ref):
    s = bfs_ref[0]
    dd = bfs_ref[1]
    lab_s = lab16_ref[pl.ds(s, 1), :]
    lab_d = lab16_ref[pl.ds(dd, 1), :]
    wg1p = wg1p_ref[...]
    g_s = jnp.dot(lab_s, wg1p, preferred_element_type=jnp.float32)
    g_d = jnp.dot(lab_d, wg1p, preferred_element_type=jnp.float32)
    g_sd = jnp.dot(lab_s + lab_d, wg1p, preferred_element_type=jnp.float32)

    lane16 = lax.broadcasted_iota(jnp.int32, (1, 16), 1)
    stop16 = jnp.where(lane16 == 10, 1.0, 0.0)
    s2 = sums_ref[3:4, :] + jnp.maximum(
        jnp.dot(stop16, wg1p, preferred_element_type=jnp.float32), 0.0)
    n2 = 10001.0
    mean_h2 = (s2 - jnp.maximum(g_d, 0.0) + jnp.maximum(g_sd, 0.0)) / n2
    h2s = jnp.where(s == dd, jnp.maximum(g_sd, 0.0), jnp.maximum(g_s, 0.0))
    ht_head = jnp.dot(mean_h2 + h2s / n2, wg2_ref[...],
                      preferred_element_type=jnp.float32)

    zpocket = jnp.dot((sums_ref[2:3, :] + sums_ref[1:2, :]) / float(N),
                      wp2_ref[...], preferred_element_type=jnp.float32)
    hinit_head = sums_ref[0:1, :] / float(N)
    sumlab = sums_ref[4:5, :]
    hinit_tail = sumlab / float(N)
    lane128 = lax.broadcasted_iota(jnp.int32, (1, 128), 1)
    ht_tail = (sumlab + jnp.where(lane128 == 10, 1.0, 0.0)) / n2
    logrow = sums_ref[5:6, :]
    out_ref[...] = jnp.concatenate(
        [logrow, hinit_head, hinit_tail, ht_head, ht_tail, zpocket,
         jnp.zeros((2, 128), jnp.float32)], axis=0)


def _comb(src, dst, epad):
    e = src.shape[0]
    s = jnp.concatenate([src, jnp.zeros((epad - e,), jnp.int32)])
    d = jnp.concatenate([dst, jnp.full((epad - e,), N, jnp.int32)])
    return jnp.stack([s.reshape(-1, CH), d.reshape(-1, CH)],
                     axis=1).reshape(-1, CH)


def kernel(x_p, edge_index_p, x_l, edge_index_l, bfs_index,
           Wp1, Wp2, Wl1, Wl2, Wg1, Wg2, Wf, bf):
    f32 = jnp.float32
    combp = _comb(edge_index_p[0], edge_index_p[1], EPP)
    combl = _comb(edge_index_l[0], edge_index_l[1], ELP)

    xp_pad = jnp.pad(x_p, ((0, NPAD - N), (0, 0)))
    xp144 = jnp.pad(
        jnp.concatenate([x_p, jnp.ones((N, 1), f32)], axis=1),
        ((0, NPAD - N), (0, WP - HID - 1)))
    xl16 = jnp.pad(
        jnp.concatenate([x_l, jnp.ones((N, 1), f32)], axis=1),
        ((0, NPAD - N), (0, 0)))
    lab16 = jnp.pad(x_l[:, 4:], ((0, NPAD - N), (0, 5)))
    zrow144 = jnp.zeros((ROWS_PER_TILE, WP), f32)
    zrow128 = jnp.zeros((ROWS_PER_TILE, HID), f32)
    zrow16 = jnp.zeros((ROWS_PER_TILE, 16), f32)

    aggp2 = _sc_stage_a1(xp144, combp, zrow144)
    aggl2 = _sc_stage_a2(xl16, combl, zrow16)
    aggp_f = [aggp2[0, :, :HID], aggp2[1, :, :HID]]
    degp = [aggp2[0, :, HID:HID + 16], aggp2[1, :, HID:HID + 16]]

    wl1p = jnp.pad(Wl1, ((0, 1), (0, 0)))
    row_spec = pl.BlockSpec((BR, HID), lambda i: (i, 0))
    row16_spec = pl.BlockSpec((BR, 16), lambda i: (i, 0))
    w_spec = pl.BlockSpec((HID, HID), lambda i: (0, 0))
    hp, z1l, invp16, invl16 = pl.pallas_call(
        _tc_stage_b,
        grid=(GRID,),
        in_specs=[row_spec, row_spec, row_spec, row16_spec, row16_spec,
                  row16_spec, row16_spec, row16_spec, w_spec,
                  pl.BlockSpec((16, HID), lambda i: (0, 0))],
        out_specs=[row_spec, row_spec, row16_spec, row16_spec],
        out_shape=[
            jax.ShapeDtypeStruct((NPAD, HID), f32),
            jax.ShapeDtypeStruct((NPAD, HID), f32),
            jax.ShapeDtypeStruct((NPAD, 16), f32),
            jax.ShapeDtypeStruct((NPAD, 16), f32),
        ],
    )(xp_pad, aggp_f[0], aggp_f[1], degp[0], degp[1],
      xl16, aggl2[0], aggl2[1], Wp1, wl1p)

    aggl2p, w2 = _sc_stage_c(
        z1l, invp16, combp, combl, zrow128, zrow16)

    wf16 = jnp.pad(Wf, ((0, 0), (0, 5)))
    bf16 = jnp.pad(bf, (0, 5)).reshape(1, 16)
    wg1p = jnp.pad(Wg1, ((0, 5), (0, 0)))
    sums = pl.pallas_call(
        _tc_stage_d,
        grid=(GRID,),
        in_specs=[row_spec, row_spec, row_spec, row_spec, row16_spec,
                  row16_spec, row16_spec, row16_spec, w_spec,
                  pl.BlockSpec((HID, 16), lambda i: (0, 0)),
                  pl.BlockSpec((1, 16), lambda i: (0, 0)),
                  pl.BlockSpec((16, HID), lambda i: (0, 0))],
        out_specs=pl.BlockSpec((8, 128), lambda i: (0, 0)),
        out_shape=jax.ShapeDtypeStruct((8, 128), f32),
    )(hp, z1l, aggl2p[0], aggl2p[1], invl16, lab16, w2[0], w2[1],
      Wl2, wf16, bf16, wg1p)

    outm = pl.pallas_call(
        _tc_stage_d2,
        in_specs=[pl.BlockSpec(memory_space=pltpu.VMEM),
                  pl.BlockSpec(memory_space=pltpu.VMEM),
                  pl.BlockSpec(memory_space=pltpu.VMEM),
                  pl.BlockSpec(memory_space=pltpu.VMEM),
                  pl.BlockSpec(memory_space=pltpu.VMEM),
                  pl.BlockSpec(memory_space=pltpu.SMEM)],
        out_specs=pl.BlockSpec(memory_space=pltpu.VMEM),
        out_shape=jax.ShapeDtypeStruct((8, 128), f32),
    )(sums, lab16, Wp2, Wg2, wg1p, bfs_index[0])

    return jnp.concatenate([outm[0, 0:1], outm[1], outm[2, :11], outm[3],
                            outm[4, :11], outm[5]])


# speedup vs baseline: 1.2667x; 1.2667x over previous
import functools

import jax
import jax.numpy as jnp
from jax import lax
from jax.experimental import pallas as pl
from jax.experimental.pallas import tpu as pltpu
from jax.experimental.pallas import tpu_sc as plsc

N = 10000
NPAD = 10240
EP = 320000
EL = 160000
HID = 128
WP = 144
NC, NS = 2, 16
NW = NC * NS
CH = 128
G = 8
CP0, CP1 = 112, 48
CL0, CL1 = 56, 24
CP0C, CP1C = 112, 48
CL0C, CL1C = 56, 24
EPP = (CP0 + CP1) * NS * CH // 2 * 2
ELP = (CL0 + CL1) * NS * CH // 2 * 2
EPP = 2560 * CH
ELP = 1280 * CH
ROWS_PER_TILE = NPAD // NS

_mesh = plsc.VectorSubcoreMesh(core_axis_name="c", subcore_axis_name="s")


def _stream_sync(table, acc, comb, base_row, nchunks, idxbuf, rowbuf,
                 goff, soff):
    def group(g, _):
        pltpu.sync_copy(comb.at[pl.ds(base_row + g * 2 * G, 2 * G)], idxbuf)

        def chunk(r, _):
            pltpu.sync_copy(table.at[idxbuf.at[2 * r + goff]], rowbuf)
            pltpu.sync_copy(rowbuf, acc.at[idxbuf.at[2 * r + soff]], add=True)
            return 0
        lax.fori_loop(0, G, chunk, 0)
        return 0
    lax.fori_loop(0, nchunks // G, group, 0)


def _stream_sync_deg(table, acc, deg, comb, base_row, nchunks, idxbuf, rowbuf,
                     onesbuf):
    def group(g, _):
        pltpu.sync_copy(comb.at[pl.ds(base_row + g * 2 * G, 2 * G)], idxbuf)

        def chunk(r, _):
            pltpu.sync_copy(table.at[idxbuf.at[2 * r]], rowbuf)
            pltpu.sync_copy(rowbuf, acc.at[idxbuf.at[2 * r + 1]], add=True)
            pltpu.sync_copy(onesbuf, deg.at[idxbuf.at[2 * r + 1]], add=True)
            return 0
        lax.fori_loop(0, G, chunk, 0)
        return 0
    lax.fori_loop(0, nchunks // G, group, 0)


@functools.partial(
    pl.kernel,
    out_type=[
        jax.ShapeDtypeStruct((NC, NPAD, HID), jnp.bfloat16),
        jax.ShapeDtypeStruct((NC, NPAD, 16), jnp.float32),
        jax.ShapeDtypeStruct((NC, NPAD, 16), jnp.float32),
    ],
    mesh=_mesh,
    compiler_params=pltpu.CompilerParams(use_tc_tiling_on_sc=False),
    scratch_types=[
        pltpu.VMEM((2 * G, CH), jnp.int32),
        pltpu.VMEM((CH, HID), jnp.bfloat16),
        pltpu.VMEM((CH, 16), jnp.float32),
        pltpu.VMEM((CH, 16), jnp.float32),
        pltpu.VMEM_SHARED((NPAD, HID), jnp.bfloat16),
        pltpu.VMEM_SHARED((NPAD, 16), jnp.float32),
        pltpu.VMEM_SHARED((NPAD, 16), jnp.float32),
    ],
)
def _sc_stage_a(xpb, xl16, combp, combl, zrowb, zrow16, ones16,
                aggp_out, degp_out, aggl_out,
                idxbuf, rowbufb, rowbuf16, onesbuf, accp, degacc, accl):
    cid = lax.axis_index("c")
    sid = lax.axis_index("s")
    sl = pl.ds(sid * ROWS_PER_TILE, ROWS_PER_TILE)

    pltpu.sync_copy(ones16, onesbuf)
    pltpu.sync_copy(zrowb, accp.at[sl])
    pltpu.sync_copy(zrow16, degacc.at[sl])
    pltpu.sync_copy(zrow16, accl.at[sl])
    plsc.subcore_barrier()

    cntp = CP0 + cid * (CP1 - CP0)
    basep = cid * (NS * CP0) + sid * cntp
    cntl = CL0 + cid * (CL1 - CL0)
    basel = cid * (NS * CL0) + sid * cntl
    _stream_sync_deg(xpb, accp, degacc, combp, basep * 2, cntp, idxbuf,
                     rowbufb, onesbuf)
    _stream_sync(xl16, accl, combl, basel * 2, cntl, idxbuf, rowbuf16,
                 goff=0, soff=1)

    plsc.subcore_barrier()
    pltpu.sync_copy(accp.at[sl], aggp_out.at[cid, sl])
    pltpu.sync_copy(degacc.at[sl], degp_out.at[cid, sl])
    pltpu.sync_copy(accl.at[sl], aggl_out.at[cid, sl])


@functools.partial(
    pl.kernel,
    out_type=[
        jax.ShapeDtypeStruct((NC, NPAD, HID), jnp.float32),
        jax.ShapeDtypeStruct((NC, NPAD, 16), jnp.float32),
    ],
    mesh=_mesh,
    compiler_params=pltpu.CompilerParams(use_tc_tiling_on_sc=False),
    scratch_types=[
        pltpu.VMEM((2 * G, CH), jnp.int32),
        pltpu.VMEM((CH, HID), jnp.float32),
        pltpu.VMEM((CH, 16), jnp.float32),
        pltpu.VMEM_SHARED((NPAD, HID), jnp.float32),
        pltpu.VMEM_SHARED((NPAD, 16), jnp.float32),
    ],
)
def _sc_stage_c(z1l, invp16, combp, combl, zrow128, zrow16,
                aggl2_out, w_out,
                idxbuf, rowbuf, rowbuf16, accl2, wacc):
    cid = lax.axis_index("c")
    sid = lax.axis_index("s")
    sl = pl.ds(sid * ROWS_PER_TILE, ROWS_PER_TILE)

    pltpu.sync_copy(zrow128, accl2.at[sl])
    pltpu.sync_copy(zrow16, wacc.at[sl])
    plsc.subcore_barrier()

    cntp = CP0C + cid * (CP1C - CP0C)
    basep = cid * (NS * CP0C) + sid * cntp
    cntl = CL0C + cid * (CL1C - CL0C)
    basel = cid * (NS * CL0C) + sid * cntl
    _stream_sync(z1l, accl2, combl, basel * 2, cntl, idxbuf, rowbuf,
                 goff=0, soff=1)
    _stream_sync(invp16, wacc, combp, basep * 2, cntp, idxbuf, rowbuf16,
                 goff=1, soff=0)

    plsc.subcore_barrier()
    pltpu.sync_copy(accl2.at[sl], aggl2_out.at[cid, sl])
    pltpu.sync_copy(wacc.at[sl], w_out.at[cid, sl])


BR = 1280
GRID = NPAD // BR


def _tc_stage_b(xp_ref, aggp0_ref, aggp1_ref, degp0_ref, degp1_ref,
                xl16_ref, aggl0_ref, aggl1_ref, wp1_ref, wl1p_ref,
                hp_ref, z1l_ref, invp16_ref, invl16_ref):
    i = pl.program_id(0)
    rows = lax.broadcasted_iota(jnp.int32, (BR, 1), 0) + i * BR
    mask = rows < N

    degp = degp0_ref[:, :1] + degp1_ref[:, :1]
    invp = jnp.where(mask, 1.0 / jnp.maximum(degp, 1.0), 0.0)
    aggp = aggp0_ref[...] + aggp1_ref[...]
    hp = jnp.maximum(jnp.dot(aggp * invp + xp_ref[...], wp1_ref[...],
                             preferred_element_type=jnp.float32), 0.0)
    hp_ref[...] = jnp.where(mask, hp, 0.0)

    aggl = aggl0_ref[...] + aggl1_ref[...]
    degl = aggl[:, 15:16]
    invl = jnp.where(mask, 1.0 / jnp.maximum(degl, 1.0), 0.0)
    z1 = jnp.maximum(jnp.dot(aggl * invl + xl16_ref[...], wl1p_ref[...],
                             preferred_element_type=jnp.float32), 0.0)
    z1l_ref[...] = jnp.where(mask, z1, 0.0)

    invp16_ref[...] = jnp.broadcast_to(invp, (BR, 16))
    invl16_ref[...] = jnp.broadcast_to(invl, (BR, 16))


def _tc_stage_d(hp_ref, z1l_ref, aggl20_ref, aggl21_ref, invl16_ref,
                lab16_ref, w0_ref, w1_ref, wl2_ref, wf16_ref, bf16_ref,
                wg1p_ref, sums_ref):
    i = pl.program_id(0)
    rows = lax.broadcasted_iota(jnp.int32, (BR, 1), 0) + i * BR
    mask = rows < N

    invl = invl16_ref[:, :1]
    zv = jnp.dot((aggl20_ref[...] + aggl21_ref[...]) * invl + z1l_ref[...],
                 wl2_ref[...], preferred_element_type=jnp.float32)

    lab = lab16_ref[...]
    logits = jnp.dot(zv, wf16_ref[...], preferred_element_type=jnp.float32) \
        + bf16_ref[...]
    lane = lax.broadcasted_iota(jnp.int32, (BR, 16), 1)
    lmask = lane < 10
    m = jnp.max(jnp.where(lmask, logits, -3e38), axis=1, keepdims=True)
    p = jnp.where(lmask, jnp.exp(logits - m), 0.0)
    val = jnp.sum(p * lab, axis=1, keepdims=True) / jnp.sum(p, axis=1, keepdims=True)
    logterm = jnp.where(mask, jnp.log(val + 1e-12), 0.0)

    hp = hp_ref[...]
    wrow = w0_ref[:, :1] + w1_ref[:, :1]
    relu_g = jnp.maximum(jnp.dot(lab, wg1p_ref[...],
                                 preferred_element_type=jnp.float32), 0.0)

    r_zv = jnp.sum(zv, axis=0, keepdims=True)
    r_hp = jnp.sum(hp, axis=0, keepdims=True)
    r_wh = jnp.sum(wrow * hp, axis=0, keepdims=True)
    r_rg = jnp.sum(relu_g, axis=0, keepdims=True)
    r_lab = jnp.concatenate(
        [jnp.sum(lab, axis=0, keepdims=True), jnp.zeros((1, 112), jnp.float32)],
        axis=1)
    lane128 = lax.broadcasted_iota(jnp.int32, (1, 128), 1)
    r_log = jnp.where(lane128 == 0, jnp.sum(logterm), 0.0)
    add = jnp.concatenate(
        [r_zv, r_hp, r_wh, r_rg, r_lab, r_log, jnp.zeros((2, 128), jnp.float32)],
        axis=0)

    @pl.when(i == 0)
    def _():
        sums_ref[...] = jnp.zeros((8, 128), jnp.float32)

    sums_ref[...] += add


def _tc_stage_d2(sums_ref, lab16_ref, wp2_ref, wg2_ref, wg1p_ref, bfs_ref,
                 out_ref):
    s = bfs_ref[0]
    dd = bfs_ref[1]
    lab_s = lab16_ref[pl.ds(s, 1), :]
    lab_d = lab16_ref[pl.ds(dd, 1), :]
    wg1p = wg1p_ref[...]
    g_s = jnp.dot(lab_s, wg1p, preferred_element_type=jnp.float32)
    g_d = jnp.dot(lab_d, wg1p, preferred_element_type=jnp.float32)
    g_sd = jnp.dot(lab_s + lab_d, wg1p, preferred_element_type=jnp.float32)

    lane16 = lax.broadcasted_iota(jnp.int32, (1, 16), 1)
    stop16 = jnp.where(lane16 == 10, 1.0, 0.0)
    s2 = sums_ref[3:4, :] + jnp.maximum(
        jnp.dot(stop16, wg1p, preferred_element_type=jnp.float32), 0.0)
    n2 = 10001.0
    mean_h2 = (s2 - jnp.maximum(g_d, 0.0) + jnp.maximum(g_sd, 0.0)) / n2
    h2s = jnp.where(s == dd, jnp.maximum(g_sd, 0.0), jnp.maximum(g_s, 0.0))
    ht_head = jnp.dot(mean_h2 + h2s / n2, wg2_ref[...],
                      preferred_element_type=jnp.float32)

    zpocket = jnp.dot((sums_ref[2:3, :] + sums_ref[1:2, :]) / float(N),
                      wp2_ref[...], preferred_element_type=jnp.float32)
    hinit_head = sums_ref[0:1, :] / float(N)
    sumlab = sums_ref[4:5, :]
    hinit_tail = sumlab / float(N)
    lane128 = lax.broadcasted_iota(jnp.int32, (1, 128), 1)
    ht_tail = (sumlab + jnp.where(lane128 == 10, 1.0, 0.0)) / n2
    logrow = sums_ref[5:6, :]
    out_ref[...] = jnp.concatenate(
        [logrow, hinit_head, hinit_tail, ht_head, ht_tail, zpocket,
         jnp.zeros((2, 128), jnp.float32)], axis=0)


def _comb(src, dst, epad):
    e = src.shape[0]
    s = jnp.concatenate([src, jnp.zeros((epad - e,), jnp.int32)])
    d = jnp.concatenate([dst, jnp.full((epad - e,), N, jnp.int32)])
    return jnp.stack([s.reshape(-1, CH), d.reshape(-1, CH)],
                     axis=1).reshape(-1, CH)


def kernel(x_p, edge_index_p, x_l, edge_index_l, bfs_index,
           Wp1, Wp2, Wl1, Wl2, Wg1, Wg2, Wf, bf):
    f32 = jnp.float32
    combp = _comb(edge_index_p[0], edge_index_p[1], EPP)
    combl = _comb(edge_index_l[0], edge_index_l[1], ELP)

    xp_pad = jnp.pad(x_p, ((0, NPAD - N), (0, 0)))
    xpb = xp_pad.astype(jnp.bfloat16)
    xl16 = jnp.pad(
        jnp.concatenate([x_l, jnp.ones((N, 1), f32)], axis=1),
        ((0, NPAD - N), (0, 0)))
    lab16 = jnp.pad(x_l[:, 4:], ((0, NPAD - N), (0, 5)))
    zrowb = jnp.zeros((ROWS_PER_TILE, HID), jnp.bfloat16)
    zrow128 = jnp.zeros((ROWS_PER_TILE, HID), f32)
    zrow16 = jnp.zeros((ROWS_PER_TILE, 16), f32)
    ones16 = jnp.ones((CH, 16), f32)

    aggp2, degp2, aggl2 = _sc_stage_a(
        xpb, xl16, combp, combl, zrowb, zrow16, ones16)
    aggp_f = [aggp2[0].astype(f32), aggp2[1].astype(f32)]
    degp = [degp2[0], degp2[1]]

    wl1p = jnp.pad(Wl1, ((0, 1), (0, 0)))
    row_spec = pl.BlockSpec((BR, HID), lambda i: (i, 0))
    row16_spec = pl.BlockSpec((BR, 16), lambda i: (i, 0))
    w_spec = pl.BlockSpec((HID, HID), lambda i: (0, 0))
    hp, z1l, invp16, invl16 = pl.pallas_call(
        _tc_stage_b,
        grid=(GRID,),
        in_specs=[row_spec, row_spec, row_spec, row16_spec, row16_spec,
                  row16_spec, row16_spec, row16_spec, w_spec,
                  pl.BlockSpec((16, HID), lambda i: (0, 0))],
        out_specs=[row_spec, row_spec, row16_spec, row16_spec],
        out_shape=[
            jax.ShapeDtypeStruct((NPAD, HID), f32),
            jax.ShapeDtypeStruct((NPAD, HID), f32),
            jax.ShapeDtypeStruct((NPAD, 16), f32),
            jax.ShapeDtypeStruct((NPAD, 16), f32),
        ],
    )(xp_pad, aggp_f[0], aggp_f[1], degp[0], degp[1],
      xl16, aggl2[0], aggl2[1], Wp1, wl1p)

    aggl2p, w2 = _sc_stage_c(
        z1l, invp16, combp, combl, zrow128, zrow16)

    wf16 = jnp.pad(Wf, ((0, 0), (0, 5)))
    bf16 = jnp.pad(bf, (0, 5)).reshape(1, 16)
    wg1p = jnp.pad(Wg1, ((0, 5), (0, 0)))
    sums = pl.pallas_call(
        _tc_stage_d,
        grid=(GRID,),
        in_specs=[row_spec, row_spec, row_spec, row_spec, row16_spec,
                  row16_spec, row16_spec, row16_spec, w_spec,
                  pl.BlockSpec((HID, 16), lambda i: (0, 0)),
                  pl.BlockSpec((1, 16), lambda i: (0, 0)),
                  pl.BlockSpec((16, HID), lambda i: (0, 0))],
        out_specs=pl.BlockSpec((8, 128), lambda i: (0, 0)),
        out_shape=jax.ShapeDtypeStruct((8, 128), f32),
    )(hp, z1l, aggl2p[0], aggl2p[1], invl16, lab16, w2[0], w2[1],
      Wl2, wf16, bf16, wg1p)

    outm = pl.pallas_call(
        _tc_stage_d2,
        in_specs=[pl.BlockSpec(memory_space=pltpu.VMEM),
                  pl.BlockSpec(memory_space=pltpu.VMEM),
                  pl.BlockSpec(memory_space=pltpu.VMEM),
                  pl.BlockSpec(memory_space=pltpu.VMEM),
                  pl.BlockSpec(memory_space=pltpu.VMEM),
                  pl.BlockSpec(memory_space=pltpu.SMEM)],
        out_specs=pl.BlockSpec(memory_space=pltpu.VMEM),
        out_shape=jax.ShapeDtypeStruct((8, 128), f32),
    )(sums, lab16, Wp2, Wg2, wg1p, bfs_index[0])

    return jnp.concatenate([outm[0, 0:1], outm[1], outm[2, :11], outm[3],
                            outm[4, :11], outm[5]])


# speedup vs baseline: 1.4186x; 1.1199x over previous
import functools

import jax
import jax.numpy as jnp
from jax import lax
from jax.experimental import pallas as pl
from jax.experimental.pallas import tpu as pltpu
from jax.experimental.pallas import tpu_sc as plsc

N = 10000
NPAD = 10240
EP = 320000
EL = 160000
HID = 128
WP = 144
NC, NS = 2, 16
NW = NC * NS
CH = 128
G = 8
CP0, CP1 = 112, 48
CL0, CL1 = 56, 24
CP0C, CP1C = 112, 48
CL0C, CL1C = 56, 24
EPP = (CP0 + CP1) * NS * CH // 2 * 2
ELP = (CL0 + CL1) * NS * CH // 2 * 2
EPP = 2560 * CH
ELP = 1280 * CH
ROWS_PER_TILE = NPAD // NS

_mesh = plsc.VectorSubcoreMesh(core_axis_name="c", subcore_axis_name="s")


def _stream_sync(table, acc, comb, base_row, nchunks, idxbuf, rowbuf,
                 goff, soff):
    def group(g, _):
        pltpu.sync_copy(comb.at[pl.ds(base_row + g * 2 * G, 2 * G)], idxbuf)

        def chunk(r, _):
            pltpu.sync_copy(table.at[idxbuf.at[2 * r + goff]], rowbuf)
            pltpu.sync_copy(rowbuf, acc.at[idxbuf.at[2 * r + soff]], add=True)
            return 0
        lax.fori_loop(0, G, chunk, 0)
        return 0
    lax.fori_loop(0, nchunks // G, group, 0)


def _stream_sync_deg(table, acc, deg, comb, base_row, nchunks, idxbuf, rowbuf,
                     onesbuf):
    def group(g, _):
        pltpu.sync_copy(comb.at[pl.ds(base_row + g * 2 * G, 2 * G)], idxbuf)

        def chunk(r, _):
            pltpu.sync_copy(table.at[idxbuf.at[2 * r]], rowbuf)
            pltpu.sync_copy(rowbuf, acc.at[idxbuf.at[2 * r + 1]], add=True)
            pltpu.sync_copy(onesbuf, deg.at[idxbuf.at[2 * r + 1]], add=True)
            return 0
        lax.fori_loop(0, G, chunk, 0)
        return 0
    lax.fori_loop(0, nchunks // G, group, 0)


@functools.partial(
    pl.kernel,
    out_type=[
        jax.ShapeDtypeStruct((NC, NPAD, HID), jnp.bfloat16),
        jax.ShapeDtypeStruct((NC, NPAD, 16), jnp.float32),
        jax.ShapeDtypeStruct((NC, NPAD, 16), jnp.float32),
    ],
    mesh=_mesh,
    compiler_params=pltpu.CompilerParams(use_tc_tiling_on_sc=False),
    scratch_types=[
        pltpu.VMEM((2 * G, CH), jnp.int32),
        pltpu.VMEM((CH, HID), jnp.bfloat16),
        pltpu.VMEM((CH, 16), jnp.float32),
        pltpu.VMEM((CH, 16), jnp.float32),
        pltpu.VMEM_SHARED((NPAD, HID), jnp.bfloat16),
        pltpu.VMEM_SHARED((NPAD, 16), jnp.float32),
        pltpu.VMEM_SHARED((NPAD, 16), jnp.float32),
    ],
)
def _sc_stage_a(xpb, xl16, combp, combl, zrowb, zrow16, ones16,
                aggp_out, degp_out, aggl_out,
                idxbuf, rowbufb, rowbuf16, onesbuf, accp, degacc, accl):
    cid = lax.axis_index("c")
    sid = lax.axis_index("s")
    sl = pl.ds(sid * ROWS_PER_TILE, ROWS_PER_TILE)

    pltpu.sync_copy(ones16, onesbuf)
    pltpu.sync_copy(zrowb, accp.at[sl])
    pltpu.sync_copy(zrow16, degacc.at[sl])
    pltpu.sync_copy(zrow16, accl.at[sl])
    plsc.subcore_barrier()

    cntp = CP0 + cid * (CP1 - CP0)
    basep = cid * (NS * CP0) + sid * cntp
    cntl = CL0 + cid * (CL1 - CL0)
    basel = cid * (NS * CL0) + sid * cntl
    _stream_sync_deg(xpb, accp, degacc, combp, basep * 2, cntp, idxbuf,
                     rowbufb, onesbuf)
    _stream_sync(xl16, accl, combl, basel * 2, cntl, idxbuf, rowbuf16,
                 goff=0, soff=1)

    plsc.subcore_barrier()
    pltpu.sync_copy(accp.at[sl], aggp_out.at[cid, sl])
    pltpu.sync_copy(degacc.at[sl], degp_out.at[cid, sl])
    pltpu.sync_copy(accl.at[sl], aggl_out.at[cid, sl])


@functools.partial(
    pl.kernel,
    out_type=[
        jax.ShapeDtypeStruct((NC, NPAD, HID), jnp.bfloat16),
        jax.ShapeDtypeStruct((NC, NPAD, 16), jnp.float32),
    ],
    mesh=_mesh,
    compiler_params=pltpu.CompilerParams(use_tc_tiling_on_sc=False),
    scratch_types=[
        pltpu.VMEM((2 * G, CH), jnp.int32),
        pltpu.VMEM((CH, HID), jnp.bfloat16),
        pltpu.VMEM((CH, 16), jnp.float32),
        pltpu.VMEM_SHARED((NPAD, HID), jnp.bfloat16),
        pltpu.VMEM_SHARED((NPAD, 16), jnp.float32),
    ],
)
def _sc_stage_c(z1l, invp16, combp, combl, zrowb, zrow16,
                aggl2_out, w_out,
                idxbuf, rowbuf, rowbuf16, accl2, wacc):
    cid = lax.axis_index("c")
    sid = lax.axis_index("s")
    sl = pl.ds(sid * ROWS_PER_TILE, ROWS_PER_TILE)

    pltpu.sync_copy(zrowb, accl2.at[sl])
    pltpu.sync_copy(zrow16, wacc.at[sl])
    plsc.subcore_barrier()

    cntp = CP0C + cid * (CP1C - CP0C)
    basep = cid * (NS * CP0C) + sid * cntp
    cntl = CL0C + cid * (CL1C - CL0C)
    basel = cid * (NS * CL0C) + sid * cntl
    _stream_sync(z1l, accl2, combl, basel * 2, cntl, idxbuf, rowbuf,
                 goff=0, soff=1)
    _stream_sync(invp16, wacc, combp, basep * 2, cntp, idxbuf, rowbuf16,
                 goff=1, soff=0)

    plsc.subcore_barrier()
    pltpu.sync_copy(accl2.at[sl], aggl2_out.at[cid, sl])
    pltpu.sync_copy(wacc.at[sl], w_out.at[cid, sl])


BR = 1280
GRID = NPAD // BR


def _tc_stage_b(xp_ref, aggp0_ref, aggp1_ref, degp0_ref, degp1_ref,
                xl16_ref, aggl0_ref, aggl1_ref, wp1_ref, wl1p_ref,
                hp_ref, z1l_ref, invp16_ref, invl16_ref):
    i = pl.program_id(0)
    rows = lax.broadcasted_iota(jnp.int32, (BR, 1), 0) + i * BR
    mask = rows < N

    degp = degp0_ref[:, :1] + degp1_ref[:, :1]
    invp = jnp.where(mask, 1.0 / jnp.maximum(degp, 1.0), 0.0)
    aggp = aggp0_ref[...].astype(jnp.float32) + aggp1_ref[...].astype(jnp.float32)
    hp = jnp.maximum(jnp.dot(aggp * invp + xp_ref[...], wp1_ref[...],
                             preferred_element_type=jnp.float32), 0.0)
    hp_ref[...] = jnp.where(mask, hp, 0.0)

    aggl = aggl0_ref[...] + aggl1_ref[...]
    degl = aggl[:, 15:16]
    invl = jnp.where(mask, 1.0 / jnp.maximum(degl, 1.0), 0.0)
    z1 = jnp.maximum(jnp.dot(aggl * invl + xl16_ref[...], wl1p_ref[...],
                             preferred_element_type=jnp.float32), 0.0)
    z1l_ref[...] = jnp.where(mask, z1, 0.0).astype(jnp.bfloat16)

    invp16_ref[...] = jnp.broadcast_to(invp, (BR, 16))
    invl16_ref[...] = jnp.broadcast_to(invl, (BR, 16))


def _tc_stage_d(hp_ref, z1l_ref, aggl20_ref, aggl21_ref, invl16_ref,
                lab16_ref, w0_ref, w1_ref, wl2_ref, wf16_ref, bf16_ref,
                wg1p_ref, sums_ref):
    i = pl.program_id(0)
    rows = lax.broadcasted_iota(jnp.int32, (BR, 1), 0) + i * BR
    mask = rows < N

    invl = invl16_ref[:, :1]
    aggl2 = aggl20_ref[...].astype(jnp.float32) + aggl21_ref[...].astype(jnp.float32)
    z1l = z1l_ref[...].astype(jnp.float32)
    zv = jnp.dot(aggl2 * invl + z1l, wl2_ref[...],
                 preferred_element_type=jnp.float32)

    lab = lab16_ref[...]
    logits = jnp.dot(zv, wf16_ref[...], preferred_element_type=jnp.float32) \
        + bf16_ref[...]
    lane = lax.broadcasted_iota(jnp.int32, (BR, 16), 1)
    lmask = lane < 10
    m = jnp.max(jnp.where(lmask, logits, -3e38), axis=1, keepdims=True)
    p = jnp.where(lmask, jnp.exp(logits - m), 0.0)
    val = jnp.sum(p * lab, axis=1, keepdims=True) / jnp.sum(p, axis=1, keepdims=True)
    logterm = jnp.where(mask, jnp.log(val + 1e-12), 0.0)

    hp = hp_ref[...]
    wrow = w0_ref[:, :1] + w1_ref[:, :1]
    relu_g = jnp.maximum(jnp.dot(lab, wg1p_ref[...],
                                 preferred_element_type=jnp.float32), 0.0)

    r_zv = jnp.sum(zv, axis=0, keepdims=True)
    r_hp = jnp.sum(hp, axis=0, keepdims=True)
    r_wh = jnp.sum(wrow * hp, axis=0, keepdims=True)
    r_rg = jnp.sum(relu_g, axis=0, keepdims=True)
    r_lab = jnp.concatenate(
        [jnp.sum(lab, axis=0, keepdims=True), jnp.zeros((1, 112), jnp.float32)],
        axis=1)
    lane128 = lax.broadcasted_iota(jnp.int32, (1, 128), 1)
    r_log = jnp.where(lane128 == 0, jnp.sum(logterm), 0.0)
    add = jnp.concatenate(
        [r_zv, r_hp, r_wh, r_rg, r_lab, r_log, jnp.zeros((2, 128), jnp.float32)],
        axis=0)

    @pl.when(i == 0)
    def _():
        sums_ref[...] = jnp.zeros((8, 128), jnp.float32)

    sums_ref[...] += add


def _tc_stage_d2(sums_ref, lab16_ref, wp2_ref, wg2_ref, wg1p_ref, bfs_ref,
                 out_ref):
    s = bfs_ref[0]
    dd = bfs_ref[1]
    lab_s = lab16_ref[pl.ds(s, 1), :]
    lab_d = lab16_ref[pl.ds(dd, 1), :]
    wg1p = wg1p_ref[...]
    g_s = jnp.dot(lab_s, wg1p, preferred_element_type=jnp.float32)
    g_d = jnp.dot(lab_d, wg1p, preferred_element_type=jnp.float32)
    g_sd = jnp.dot(lab_s + lab_d, wg1p, preferred_element_type=jnp.float32)

    lane16 = lax.broadcasted_iota(jnp.int32, (1, 16), 1)
    stop16 = jnp.where(lane16 == 10, 1.0, 0.0)
    s2 = sums_ref[3:4, :] + jnp.maximum(
        jnp.dot(stop16, wg1p, preferred_element_type=jnp.float32), 0.0)
    n2 = 10001.0
    mean_h2 = (s2 - jnp.maximum(g_d, 0.0) + jnp.maximum(g_sd, 0.0)) / n2
    h2s = jnp.where(s == dd, jnp.maximum(g_sd, 0.0), jnp.maximum(g_s, 0.0))
    ht_head = jnp.dot(mean_h2 + h2s / n2, wg2_ref[...],
                      preferred_element_type=jnp.float32)

    zpocket = jnp.dot((sums_ref[2:3, :] + sums_ref[1:2, :]) / float(N),
                      wp2_ref[...], preferred_element_type=jnp.float32)
    hinit_head = sums_ref[0:1, :] / float(N)
    sumlab = sums_ref[4:5, :]
    hinit_tail = sumlab / float(N)
    lane128 = lax.broadcasted_iota(jnp.int32, (1, 128), 1)
    ht_tail = (sumlab + jnp.where(lane128 == 10, 1.0, 0.0)) / n2
    logrow = sums_ref[5:6, :]
    out_ref[...] = jnp.concatenate(
        [logrow, hinit_head, hinit_tail, ht_head, ht_tail, zpocket,
         jnp.zeros((2, 128), jnp.float32)], axis=0)


def _comb(src, dst, epad):
    e = src.shape[0]
    s = jnp.concatenate([src, jnp.zeros((epad - e,), jnp.int32)])
    d = jnp.concatenate([dst, jnp.full((epad - e,), N, jnp.int32)])
    return jnp.stack([s.reshape(-1, CH), d.reshape(-1, CH)],
                     axis=1).reshape(-1, CH)


def kernel(x_p, edge_index_p, x_l, edge_index_l, bfs_index,
           Wp1, Wp2, Wl1, Wl2, Wg1, Wg2, Wf, bf):
    f32 = jnp.float32
    combp = _comb(edge_index_p[0], edge_index_p[1], EPP)
    combl = _comb(edge_index_l[0], edge_index_l[1], ELP)

    xp_pad = jnp.pad(x_p, ((0, NPAD - N), (0, 0)))
    xpb = xp_pad.astype(jnp.bfloat16)
    xl16 = jnp.pad(
        jnp.concatenate([x_l, jnp.ones((N, 1), f32)], axis=1),
        ((0, NPAD - N), (0, 0)))
    lab16 = jnp.pad(x_l[:, 4:], ((0, NPAD - N), (0, 5)))
    zrowb = jnp.zeros((ROWS_PER_TILE, HID), jnp.bfloat16)
    zrow16 = jnp.zeros((ROWS_PER_TILE, 16), f32)
    ones16 = jnp.ones((CH, 16), f32)

    aggp2, degp2, aggl2 = _sc_stage_a(
        xpb, xl16, combp, combl, zrowb, zrow16, ones16)
    aggp_f = [aggp2[0], aggp2[1]]
    degp = [degp2[0], degp2[1]]

    wl1p = jnp.pad(Wl1, ((0, 1), (0, 0)))
    row_spec = pl.BlockSpec((BR, HID), lambda i: (i, 0))
    row16_spec = pl.BlockSpec((BR, 16), lambda i: (i, 0))
    w_spec = pl.BlockSpec((HID, HID), lambda i: (0, 0))
    hp, z1l, invp16, invl16 = pl.pallas_call(
        _tc_stage_b,
        grid=(GRID,),
        in_specs=[row_spec, row_spec, row_spec, row16_spec, row16_spec,
                  row16_spec, row16_spec, row16_spec, w_spec,
                  pl.BlockSpec((16, HID), lambda i: (0, 0))],
        out_specs=[row_spec, row_spec, row16_spec, row16_spec],
        out_shape=[
            jax.ShapeDtypeStruct((NPAD, HID), f32),
            jax.ShapeDtypeStruct((NPAD, HID), jnp.bfloat16),
            jax.ShapeDtypeStruct((NPAD, 16), f32),
            jax.ShapeDtypeStruct((NPAD, 16), f32),
        ],
    )(xp_pad, aggp_f[0], aggp_f[1], degp[0], degp[1],
      xl16, aggl2[0], aggl2[1], Wp1, wl1p)

    aggl2p, w2 = _sc_stage_c(
        z1l, invp16, combp, combl, zrowb, zrow16)

    wf16 = jnp.pad(Wf, ((0, 0), (0, 5)))
    bf16 = jnp.pad(bf, (0, 5)).reshape(1, 16)
    wg1p = jnp.pad(Wg1, ((0, 5), (0, 0)))
    sums = pl.pallas_call(
        _tc_stage_d,
        grid=(GRID,),
        in_specs=[row_spec, row_spec, row_spec, row_spec, row16_spec,
                  row16_spec, row16_spec, row16_spec, w_spec,
                  pl.BlockSpec((HID, 16), lambda i: (0, 0)),
                  pl.BlockSpec((1, 16), lambda i: (0, 0)),
                  pl.BlockSpec((16, HID), lambda i: (0, 0))],
        out_specs=pl.BlockSpec((8, 128), lambda i: (0, 0)),
        out_shape=jax.ShapeDtypeStruct((8, 128), f32),
    )(hp, z1l, aggl2p[0], aggl2p[1], invl16, lab16, w2[0], w2[1],
      Wl2, wf16, bf16, wg1p)

    outm = pl.pallas_call(
        _tc_stage_d2,
        in_specs=[pl.BlockSpec(memory_space=pltpu.VMEM),
                  pl.BlockSpec(memory_space=pltpu.VMEM),
                  pl.BlockSpec(memory_space=pltpu.VMEM),
                  pl.BlockSpec(memory_space=pltpu.VMEM),
                  pl.BlockSpec(memory_space=pltpu.VMEM),
                  pl.BlockSpec(memory_space=pltpu.SMEM)],
        out_specs=pl.BlockSpec(memory_space=pltpu.VMEM),
        out_shape=jax.ShapeDtypeStruct((8, 128), f32),
    )(sums, lab16, Wp2, Wg2, wg1p, bfs_index[0])

    return jnp.concatenate([outm[0, 0:1], outm[1], outm[2, :11], outm[3],
                            outm[4, :11], outm[5]])


# speedup vs baseline: 1.4620x; 1.0306x over previous
import functools

import jax
import jax.numpy as jnp
from jax import lax
from jax.experimental import pallas as pl
from jax.experimental.pallas import tpu as pltpu
from jax.experimental.pallas import tpu_sc as plsc

N = 10000
NPAD = 10240
EP = 320000
EL = 160000
HID = 128
WP = 144
NC, NS = 2, 16
NW = NC * NS
CH = 128
G = 8
CP0, CP1 = 112, 48
CL0, CL1 = 56, 24
CP0C, CP1C = 112, 48
CL0C, CL1C = 56, 24
EPP = (CP0 + CP1) * NS * CH // 2 * 2
ELP = (CL0 + CL1) * NS * CH // 2 * 2
EPP = 2560 * CH
ELP = 1280 * CH
ROWS_PER_TILE = NPAD // NS

_mesh = plsc.VectorSubcoreMesh(core_axis_name="c", subcore_axis_name="s")


def _stream_sync(table, acc, comb, base_row, nchunks, idxbuf, rowbuf,
                 goff, soff):
    def group(g, _):
        pltpu.sync_copy(comb.at[pl.ds(base_row + g * 2 * G, 2 * G)], idxbuf)

        def chunk(r, _):
            pltpu.sync_copy(table.at[idxbuf.at[2 * r + goff]], rowbuf)
            pltpu.sync_copy(rowbuf, acc.at[idxbuf.at[2 * r + soff]], add=True)
            return 0
        lax.fori_loop(0, G, chunk, 0)
        return 0
    lax.fori_loop(0, nchunks // G, group, 0)


def _stream_sync_deg(table, acc, deg, comb, base_row, nchunks, idxbuf, rowbuf,
                     onesbuf):
    def group(g, _):
        pltpu.sync_copy(comb.at[pl.ds(base_row + g * 2 * G, 2 * G)], idxbuf)

        def chunk(r, _):
            pltpu.sync_copy(table.at[idxbuf.at[2 * r]], rowbuf)
            pltpu.sync_copy(rowbuf, acc.at[idxbuf.at[2 * r + 1]], add=True)
            pltpu.sync_copy(onesbuf, deg.at[idxbuf.at[2 * r + 1]], add=True)
            return 0
        lax.fori_loop(0, G, chunk, 0)
        return 0
    lax.fori_loop(0, nchunks // G, group, 0)


@functools.partial(
    pl.kernel,
    out_type=[
        jax.ShapeDtypeStruct((NC, NPAD, HID), jnp.bfloat16),
        jax.ShapeDtypeStruct((NC, NPAD, 16), jnp.float32),
        jax.ShapeDtypeStruct((NC, NPAD, 16), jnp.float32),
    ],
    mesh=_mesh,
    compiler_params=pltpu.CompilerParams(use_tc_tiling_on_sc=False),
    scratch_types=[
        pltpu.VMEM((2 * G, CH), jnp.int32),
        pltpu.VMEM((CH, HID), jnp.bfloat16),
        pltpu.VMEM((CH, 16), jnp.float32),
        pltpu.VMEM((CH, 16), jnp.float32),
        pltpu.VMEM_SHARED((NPAD, HID), jnp.bfloat16),
        pltpu.VMEM_SHARED((NPAD, 16), jnp.float32),
        pltpu.VMEM_SHARED((NPAD, 16), jnp.float32),
    ],
)
def _sc_stage_a(xpb, xl16, combp, combl, zrowb, zrow16, ones16,
                aggp_out, degp_out, aggl_out,
                idxbuf, rowbufb, rowbuf16, onesbuf, accp, degacc, accl):
    cid = lax.axis_index("c")
    sid = lax.axis_index("s")
    sl = pl.ds(sid * ROWS_PER_TILE, ROWS_PER_TILE)

    pltpu.sync_copy(ones16, onesbuf)
    pltpu.sync_copy(zrowb, accp.at[sl])
    pltpu.sync_copy(zrow16, degacc.at[sl])
    pltpu.sync_copy(zrow16, accl.at[sl])
    plsc.subcore_barrier()

    cntp = CP0 + cid * (CP1 - CP0)
    basep = cid * (NS * CP0) + sid * cntp
    cntl = CL0 + cid * (CL1 - CL0)
    basel = cid * (NS * CL0) + sid * cntl
    _stream_sync_deg(xpb, accp, degacc, combp, basep * 2, cntp, idxbuf,
                     rowbufb, onesbuf)
    _stream_sync(xl16, accl, combl, basel * 2, cntl, idxbuf, rowbuf16,
                 goff=0, soff=1)

    plsc.subcore_barrier()
    pltpu.sync_copy(accp.at[sl], aggp_out.at[cid, sl])
    pltpu.sync_copy(degacc.at[sl], degp_out.at[cid, sl])
    pltpu.sync_copy(accl.at[sl], aggl_out.at[cid, sl])


@functools.partial(
    pl.kernel,
    out_type=[
        jax.ShapeDtypeStruct((NC, NPAD, HID), jnp.bfloat16),
        jax.ShapeDtypeStruct((NC, NPAD, 16), jnp.float32),
    ],
    mesh=_mesh,
    compiler_params=pltpu.CompilerParams(use_tc_tiling_on_sc=False),
    scratch_types=[
        pltpu.VMEM((2 * G, CH), jnp.int32),
        pltpu.VMEM((CH, HID), jnp.bfloat16),
        pltpu.VMEM((CH, 16), jnp.float32),
        pltpu.VMEM_SHARED((NPAD, HID), jnp.bfloat16),
        pltpu.VMEM_SHARED((NPAD, 16), jnp.float32),
    ],
)
def _sc_stage_c(z1l, invp16, combp, combl, zrowb, zrow16,
                aggl2_out, w_out,
                idxbuf, rowbuf, rowbuf16, accl2, wacc):
    cid = lax.axis_index("c")
    sid = lax.axis_index("s")
    sl = pl.ds(sid * ROWS_PER_TILE, ROWS_PER_TILE)

    pltpu.sync_copy(zrowb, accl2.at[sl])
    pltpu.sync_copy(zrow16, wacc.at[sl])
    plsc.subcore_barrier()

    cntp = CP0C + cid * (CP1C - CP0C)
    basep = cid * (NS * CP0C) + sid * cntp
    cntl = CL0C + cid * (CL1C - CL0C)
    basel = cid * (NS * CL0C) + sid * cntl
    _stream_sync(z1l, accl2, combl, basel * 2, cntl, idxbuf, rowbuf,
                 goff=0, soff=1)
    _stream_sync(invp16, wacc, combp, basep * 2, cntp, idxbuf, rowbuf16,
                 goff=1, soff=0)

    plsc.subcore_barrier()
    pltpu.sync_copy(accl2.at[sl], aggl2_out.at[cid, sl])
    pltpu.sync_copy(wacc.at[sl], w_out.at[cid, sl])


BR = 1280
GRID = NPAD // BR


def _tc_stage_b(xp_ref, aggp0_ref, aggp1_ref, degp0_ref, degp1_ref,
                xl16_ref, aggl0_ref, aggl1_ref, wp1_ref, wl1p_ref,
                hp_ref, z1l_ref, invp16_ref, invl16_ref):
    i = pl.program_id(0)
    rows = lax.broadcasted_iota(jnp.int32, (BR, 1), 0) + i * BR
    mask = rows < N

    degp = degp0_ref[:, :1] + degp1_ref[:, :1]
    invp = jnp.where(mask, 1.0 / jnp.maximum(degp, 1.0), 0.0)
    aggp = aggp0_ref[...].astype(jnp.float32) + aggp1_ref[...].astype(jnp.float32)
    hp = jnp.maximum(jnp.dot(aggp * invp + xp_ref[...], wp1_ref[...],
                             preferred_element_type=jnp.float32), 0.0)
    hp_ref[...] = jnp.where(mask, hp, 0.0)

    aggl = aggl0_ref[...] + aggl1_ref[...]
    degl = aggl[:, 15:16]
    invl = jnp.where(mask, 1.0 / jnp.maximum(degl, 1.0), 0.0)
    z1 = jnp.maximum(jnp.dot(aggl * invl + xl16_ref[...], wl1p_ref[...],
                             preferred_element_type=jnp.float32), 0.0)
    z1l_ref[...] = jnp.where(mask, z1, 0.0).astype(jnp.bfloat16)

    invp16_ref[...] = jnp.broadcast_to(invp, (BR, 16))
    invl16_ref[...] = jnp.broadcast_to(invl, (BR, 16))


def _tc_stage_d(hp_ref, z1l_ref, aggl20_ref, aggl21_ref, invl16_ref,
                lab16_ref, w0_ref, w1_ref, wl2_ref, wf16_ref, bf16_ref,
                wg1p_ref, wp2_ref, wg2_ref, lab_any_ref, bfs_ref,
                sums_ref, outm_ref, labs_v, labd_v, dsem):
    i = pl.program_id(0)
    rows = lax.broadcasted_iota(jnp.int32, (BR, 1), 0) + i * BR
    mask = rows < N

    invl = invl16_ref[:, :1]
    aggl2 = aggl20_ref[...].astype(jnp.float32) + aggl21_ref[...].astype(jnp.float32)
    z1l = z1l_ref[...].astype(jnp.float32)
    zv = jnp.dot(aggl2 * invl + z1l, wl2_ref[...],
                 preferred_element_type=jnp.float32)

    lab = lab16_ref[...]
    logits = jnp.dot(zv, wf16_ref[...], preferred_element_type=jnp.float32) \
        + bf16_ref[...]
    lane = lax.broadcasted_iota(jnp.int32, (BR, 16), 1)
    lmask = lane < 10
    m = jnp.max(jnp.where(lmask, logits, -3e38), axis=1, keepdims=True)
    p = jnp.where(lmask, jnp.exp(logits - m), 0.0)
    val = jnp.sum(p * lab, axis=1, keepdims=True) / jnp.sum(p, axis=1, keepdims=True)
    logterm = jnp.where(mask, jnp.log(val + 1e-12), 0.0)

    hp = hp_ref[...]
    wrow = w0_ref[:, :1] + w1_ref[:, :1]
    relu_g = jnp.maximum(jnp.dot(lab, wg1p_ref[...],
                                 preferred_element_type=jnp.float32), 0.0)

    r_zv = jnp.sum(zv, axis=0, keepdims=True)
    r_hp = jnp.sum(hp, axis=0, keepdims=True)
    r_wh = jnp.sum(wrow * hp, axis=0, keepdims=True)
    r_rg = jnp.sum(relu_g, axis=0, keepdims=True)
    r_lab = jnp.concatenate(
        [jnp.sum(lab, axis=0, keepdims=True), jnp.zeros((1, 112), jnp.float32)],
        axis=1)
    lane128 = lax.broadcasted_iota(jnp.int32, (1, 128), 1)
    r_log = jnp.where(lane128 == 0, jnp.sum(logterm), 0.0)
    add = jnp.concatenate(
        [r_zv, r_hp, r_wh, r_rg, r_lab, r_log, jnp.zeros((2, 128), jnp.float32)],
        axis=0)

    @pl.when(i == 0)
    def _():
        sums_ref[...] = jnp.zeros((8, 128), jnp.float32)

    sums_ref[...] += add

    @pl.when(i == GRID - 1)
    def _():
        _final_combine(sums_ref, lab_any_ref, wp2_ref, wg2_ref, wg1p_ref,
                       bfs_ref, outm_ref, labs_v, labd_v, dsem)


def _final_combine(sums_ref, lab_any_ref, wp2_ref, wg2_ref, wg1p_ref,
                   bfs_ref, out_ref, labs_v, labd_v, dsem):
    s = bfs_ref[0]
    dd = bfs_ref[1]
    cp_s = pltpu.make_async_copy(lab_any_ref.at[pl.ds(s, 1)], labs_v, dsem)
    cp_s.start()
    cp_d = pltpu.make_async_copy(lab_any_ref.at[pl.ds(dd, 1)], labd_v, dsem)
    cp_d.start()
    cp_s.wait()
    cp_d.wait()
    lab_s = labs_v[...]
    lab_d = labd_v[...]
    wg1p = wg1p_ref[...]
    g_s = jnp.dot(lab_s, wg1p, preferred_element_type=jnp.float32)
    g_d = jnp.dot(lab_d, wg1p, preferred_element_type=jnp.float32)
    g_sd = jnp.dot(lab_s + lab_d, wg1p, preferred_element_type=jnp.float32)

    lane16 = lax.broadcasted_iota(jnp.int32, (1, 16), 1)
    stop16 = jnp.where(lane16 == 10, 1.0, 0.0)
    s2 = sums_ref[3:4, :] + jnp.maximum(
        jnp.dot(stop16, wg1p, preferred_element_type=jnp.float32), 0.0)
    n2 = 10001.0
    mean_h2 = (s2 - jnp.maximum(g_d, 0.0) + jnp.maximum(g_sd, 0.0)) / n2
    h2s = jnp.where(s == dd, jnp.maximum(g_sd, 0.0), jnp.maximum(g_s, 0.0))
    ht_head = jnp.dot(mean_h2 + h2s / n2, wg2_ref[...],
                      preferred_element_type=jnp.float32)

    zpocket = jnp.dot((sums_ref[2:3, :] + sums_ref[1:2, :]) / float(N),
                      wp2_ref[...], preferred_element_type=jnp.float32)
    hinit_head = sums_ref[0:1, :] / float(N)
    sumlab = sums_ref[4:5, :]
    hinit_tail = sumlab / float(N)
    lane128 = lax.broadcasted_iota(jnp.int32, (1, 128), 1)
    ht_tail = (sumlab + jnp.where(lane128 == 10, 1.0, 0.0)) / n2
    logrow = sums_ref[5:6, :]
    out_ref[...] = jnp.concatenate(
        [logrow, hinit_head, hinit_tail, ht_head, ht_tail, zpocket,
         jnp.zeros((2, 128), jnp.float32)], axis=0)


def _comb(src, dst, epad):
    e = src.shape[0]
    s = jnp.concatenate([src, jnp.zeros((epad - e,), jnp.int32)])
    d = jnp.concatenate([dst, jnp.full((epad - e,), N, jnp.int32)])
    return jnp.stack([s.reshape(-1, CH), d.reshape(-1, CH)],
                     axis=1).reshape(-1, CH)


def kernel(x_p, edge_index_p, x_l, edge_index_l, bfs_index,
           Wp1, Wp2, Wl1, Wl2, Wg1, Wg2, Wf, bf):
    f32 = jnp.float32
    combp = _comb(edge_index_p[0], edge_index_p[1], EPP)
    combl = _comb(edge_index_l[0], edge_index_l[1], ELP)

    xp_pad = jnp.pad(x_p, ((0, NPAD - N), (0, 0)))
    xpb = xp_pad.astype(jnp.bfloat16)
    xl16 = jnp.pad(
        jnp.concatenate([x_l, jnp.ones((N, 1), f32)], axis=1),
        ((0, NPAD - N), (0, 0)))
    lab16 = jnp.pad(x_l[:, 4:], ((0, NPAD - N), (0, 5)))
    zrowb = jnp.zeros((ROWS_PER_TILE, HID), jnp.bfloat16)
    zrow16 = jnp.zeros((ROWS_PER_TILE, 16), f32)
    ones16 = jnp.ones((CH, 16), f32)

    aggp2, degp2, aggl2 = _sc_stage_a(
        xpb, xl16, combp, combl, zrowb, zrow16, ones16)
    aggp_f = [aggp2[0], aggp2[1]]
    degp = [degp2[0], degp2[1]]

    wl1p = jnp.pad(Wl1, ((0, 1), (0, 0)))
    row_spec = pl.BlockSpec((BR, HID), lambda i: (i, 0))
    row16_spec = pl.BlockSpec((BR, 16), lambda i: (i, 0))
    w_spec = pl.BlockSpec((HID, HID), lambda i: (0, 0))
    hp, z1l, invp16, invl16 = pl.pallas_call(
        _tc_stage_b,
        grid=(GRID,),
        in_specs=[row_spec, row_spec, row_spec, row16_spec, row16_spec,
                  row16_spec, row16_spec, row16_spec, w_spec,
                  pl.BlockSpec((16, HID), lambda i: (0, 0))],
        out_specs=[row_spec, row_spec, row16_spec, row16_spec],
        out_shape=[
            jax.ShapeDtypeStruct((NPAD, HID), f32),
            jax.ShapeDtypeStruct((NPAD, HID), jnp.bfloat16),
            jax.ShapeDtypeStruct((NPAD, 16), f32),
            jax.ShapeDtypeStruct((NPAD, 16), f32),
        ],
    )(xp_pad, aggp_f[0], aggp_f[1], degp[0], degp[1],
      xl16, aggl2[0], aggl2[1], Wp1, wl1p)

    aggl2p, w2 = _sc_stage_c(
        z1l, invp16, combp, combl, zrowb, zrow16)

    wf16 = jnp.pad(Wf, ((0, 0), (0, 5)))
    bf16 = jnp.pad(bf, (0, 5)).reshape(1, 16)
    wg1p = jnp.pad(Wg1, ((0, 5), (0, 0)))
    _, outm = pl.pallas_call(
        _tc_stage_d,
        grid=(GRID,),
        in_specs=[row_spec, row_spec, row_spec, row_spec, row16_spec,
                  row16_spec, row16_spec, row16_spec, w_spec,
                  pl.BlockSpec((HID, 16), lambda i: (0, 0)),
                  pl.BlockSpec((1, 16), lambda i: (0, 0)),
                  pl.BlockSpec((16, HID), lambda i: (0, 0)),
                  w_spec, w_spec,
                  pl.BlockSpec(memory_space=pl.ANY),
                  pl.BlockSpec(memory_space=pltpu.SMEM)],
        out_specs=[pl.BlockSpec((8, 128), lambda i: (0, 0)),
                   pl.BlockSpec((8, 128), lambda i: (0, 0))],
        out_shape=[jax.ShapeDtypeStruct((8, 128), f32),
                   jax.ShapeDtypeStruct((8, 128), f32)],
        scratch_shapes=[pltpu.VMEM((1, 16), f32), pltpu.VMEM((1, 16), f32),
                        pltpu.SemaphoreType.DMA],
    )(hp, z1l, aggl2p[0], aggl2p[1], invl16, lab16, w2[0], w2[1],
      Wl2, wf16, bf16, wg1p, Wp2, Wg2, lab16, bfs_index[0])

    return jnp.concatenate([outm[0, 0:1], outm[1], outm[2, :11], outm[3],
                            outm[4, :11], outm[5]])


# speedup vs baseline: 1.4763x; 1.0098x over previous
import functools

import jax
import jax.numpy as jnp
from jax import lax
from jax.experimental import pallas as pl
from jax.experimental.pallas import tpu as pltpu
from jax.experimental.pallas import tpu_sc as plsc

N = 10000
NPAD = 10240
EP = 320000
EL = 160000
HID = 128
WP = 144
NC, NS = 2, 16
NW = NC * NS
CH = 128
G = 8
CP0, CP1 = 112, 48
CL0, CL1 = 56, 24
CP0C, CP1C = 112, 48
CL0C, CL1C = 56, 24
EPP = (CP0 + CP1) * NS * CH // 2 * 2
ELP = (CL0 + CL1) * NS * CH // 2 * 2
EPP = 2560 * CH
ELP = 1280 * CH
ROWS_PER_TILE = NPAD // NS

_mesh = plsc.VectorSubcoreMesh(core_axis_name="c", subcore_axis_name="s")


def _stream_sync(table, acc, comb, base_row, nchunks, idxbuf, rowbuf,
                 goff, soff, grp=G):
    def group(g, _):
        pltpu.sync_copy(comb.at[pl.ds(base_row + g * 2 * grp, 2 * grp)],
                        idxbuf.at[pl.ds(0, 2 * grp)])

        def chunk(r, _):
            pltpu.sync_copy(table.at[idxbuf.at[2 * r + goff]], rowbuf)
            pltpu.sync_copy(rowbuf, acc.at[idxbuf.at[2 * r + soff]], add=True)
            return 0
        lax.fori_loop(0, grp, chunk, 0)
        return 0
    lax.fori_loop(0, nchunks // grp, group, 0)


def _stream_sync_deg(table, acc, deg, comb, base_row, nchunks, idxbuf, rowbuf,
                     onesbuf, grp=G):
    def group(g, _):
        pltpu.sync_copy(comb.at[pl.ds(base_row + g * 2 * grp, 2 * grp)],
                        idxbuf.at[pl.ds(0, 2 * grp)])

        def chunk(r, _):
            pltpu.sync_copy(table.at[idxbuf.at[2 * r]], rowbuf)
            pltpu.sync_copy(rowbuf, acc.at[idxbuf.at[2 * r + 1]], add=True)
            pltpu.sync_copy(onesbuf, deg.at[idxbuf.at[2 * r + 1]], add=True)
            return 0
        lax.fori_loop(0, grp, chunk, 0)
        return 0
    lax.fori_loop(0, nchunks // grp, group, 0)


@functools.partial(
    pl.kernel,
    out_type=[
        jax.ShapeDtypeStruct((NC, NPAD, HID), jnp.bfloat16),
        jax.ShapeDtypeStruct((NC, NPAD, 16), jnp.float32),
        jax.ShapeDtypeStruct((NC, NPAD, 16), jnp.float32),
    ],
    mesh=_mesh,
    compiler_params=pltpu.CompilerParams(use_tc_tiling_on_sc=False),
    scratch_types=[
        pltpu.VMEM((4 * G, CH), jnp.int32),
        pltpu.VMEM((CH, HID), jnp.bfloat16),
        pltpu.VMEM((CH, 16), jnp.float32),
        pltpu.VMEM((CH, 16), jnp.float32),
        pltpu.VMEM_SHARED((NPAD, HID), jnp.bfloat16),
        pltpu.VMEM_SHARED((NPAD, 16), jnp.float32),
        pltpu.VMEM_SHARED((NPAD, 16), jnp.float32),
    ],
)
def _sc_stage_a(xpb, xl16, combp, combl, zrowb, zrow16, ones16,
                aggp_out, degp_out, aggl_out,
                idxbuf, rowbufb, rowbuf16, onesbuf, accp, degacc, accl):
    cid = lax.axis_index("c")
    sid = lax.axis_index("s")
    sl = pl.ds(sid * ROWS_PER_TILE, ROWS_PER_TILE)

    pltpu.sync_copy(ones16, onesbuf)
    pltpu.sync_copy(zrowb, accp.at[sl])
    pltpu.sync_copy(zrow16, degacc.at[sl])
    pltpu.sync_copy(zrow16, accl.at[sl])
    plsc.subcore_barrier()

    cntp = CP0 + cid * (CP1 - CP0)
    basep = cid * (NS * CP0) + sid * cntp
    cntl = CL0 + cid * (CL1 - CL0)
    basel = cid * (NS * CL0) + sid * cntl
    _stream_sync_deg(xpb, accp, degacc, combp, basep * 2, cntp, idxbuf,
                     rowbufb, onesbuf, grp=2 * G)
    _stream_sync(xl16, accl, combl, basel * 2, cntl, idxbuf, rowbuf16,
                 goff=0, soff=1)

    plsc.subcore_barrier()
    pltpu.sync_copy(accp.at[sl], aggp_out.at[cid, sl])
    pltpu.sync_copy(degacc.at[sl], degp_out.at[cid, sl])
    pltpu.sync_copy(accl.at[sl], aggl_out.at[cid, sl])


@functools.partial(
    pl.kernel,
    out_type=[
        jax.ShapeDtypeStruct((NC, NPAD, HID), jnp.bfloat16),
        jax.ShapeDtypeStruct((NC, NPAD, 16), jnp.float32),
    ],
    mesh=_mesh,
    compiler_params=pltpu.CompilerParams(use_tc_tiling_on_sc=False),
    scratch_types=[
        pltpu.VMEM((4 * G, CH), jnp.int32),
        pltpu.VMEM((CH, HID), jnp.bfloat16),
        pltpu.VMEM((CH, 16), jnp.float32),
        pltpu.VMEM_SHARED((NPAD, HID), jnp.bfloat16),
        pltpu.VMEM_SHARED((NPAD, 16), jnp.float32),
    ],
)
def _sc_stage_c(z1l, invp16, combp, combl, zrowb, zrow16,
                aggl2_out, w_out,
                idxbuf, rowbuf, rowbuf16, accl2, wacc):
    cid = lax.axis_index("c")
    sid = lax.axis_index("s")
    sl = pl.ds(sid * ROWS_PER_TILE, ROWS_PER_TILE)

    pltpu.sync_copy(zrowb, accl2.at[sl])
    pltpu.sync_copy(zrow16, wacc.at[sl])
    plsc.subcore_barrier()

    cntp = CP0C + cid * (CP1C - CP0C)
    basep = cid * (NS * CP0C) + sid * cntp
    cntl = CL0C + cid * (CL1C - CL0C)
    basel = cid * (NS * CL0C) + sid * cntl
    _stream_sync(z1l, accl2, combl, basel * 2, cntl, idxbuf, rowbuf,
                 goff=0, soff=1)
    _stream_sync(invp16, wacc, combp, basep * 2, cntp, idxbuf, rowbuf16,
                 goff=1, soff=0, grp=2 * G)

    plsc.subcore_barrier()
    pltpu.sync_copy(accl2.at[sl], aggl2_out.at[cid, sl])
    pltpu.sync_copy(wacc.at[sl], w_out.at[cid, sl])


BR = 1280
GRID = NPAD // BR


def _tc_stage_b(xp_ref, aggp0_ref, aggp1_ref, degp0_ref, degp1_ref,
                xl16_ref, aggl0_ref, aggl1_ref, wp1_ref, wl1p_ref,
                hp_ref, z1l_ref, invp16_ref, invl16_ref):
    i = pl.program_id(0)
    rows = lax.broadcasted_iota(jnp.int32, (BR, 1), 0) + i * BR
    mask = rows < N

    degp = degp0_ref[:, :1] + degp1_ref[:, :1]
    invp = jnp.where(mask, 1.0 / jnp.maximum(degp, 1.0), 0.0)
    aggp = aggp0_ref[...].astype(jnp.float32) + aggp1_ref[...].astype(jnp.float32)
    hp = jnp.maximum(jnp.dot(aggp * invp + xp_ref[...], wp1_ref[...],
                             preferred_element_type=jnp.float32), 0.0)
    hp_ref[...] = jnp.where(mask, hp, 0.0)

    aggl = aggl0_ref[...] + aggl1_ref[...]
    degl = aggl[:, 15:16]
    invl = jnp.where(mask, 1.0 / jnp.maximum(degl, 1.0), 0.0)
    z1 = jnp.maximum(jnp.dot(aggl * invl + xl16_ref[...], wl1p_ref[...],
                             preferred_element_type=jnp.float32), 0.0)
    z1l_ref[...] = jnp.where(mask, z1, 0.0).astype(jnp.bfloat16)

    invp16_ref[...] = jnp.broadcast_to(invp, (BR, 16))
    invl16_ref[...] = jnp.broadcast_to(invl, (BR, 16))


def _tc_stage_d(hp_ref, z1l_ref, aggl20_ref, aggl21_ref, invl16_ref,
                lab16_ref, w0_ref, w1_ref, wl2_ref, wf16_ref, bf16_ref,
                wg1p_ref, wp2_ref, wg2_ref, lab_any_ref, bfs_ref,
                sums_ref, outm_ref, labs_v, labd_v, dsem):
    i = pl.program_id(0)
    rows = lax.broadcasted_iota(jnp.int32, (BR, 1), 0) + i * BR
    mask = rows < N

    invl = invl16_ref[:, :1]
    aggl2 = aggl20_ref[...].astype(jnp.float32) + aggl21_ref[...].astype(jnp.float32)
    z1l = z1l_ref[...].astype(jnp.float32)
    zv = jnp.dot(aggl2 * invl + z1l, wl2_ref[...],
                 preferred_element_type=jnp.float32)

    lab = lab16_ref[...]
    logits = jnp.dot(zv, wf16_ref[...], preferred_element_type=jnp.float32) \
        + bf16_ref[...]
    lane = lax.broadcasted_iota(jnp.int32, (BR, 16), 1)
    lmask = lane < 10
    m = jnp.max(jnp.where(lmask, logits, -3e38), axis=1, keepdims=True)
    p = jnp.where(lmask, jnp.exp(logits - m), 0.0)
    val = jnp.sum(p * lab, axis=1, keepdims=True) / jnp.sum(p, axis=1, keepdims=True)
    logterm = jnp.where(mask, jnp.log(val + 1e-12), 0.0)

    hp = hp_ref[...]
    wrow = w0_ref[:, :1] + w1_ref[:, :1]
    relu_g = jnp.maximum(jnp.dot(lab, wg1p_ref[...],
                                 preferred_element_type=jnp.float32), 0.0)

    r_zv = jnp.sum(zv, axis=0, keepdims=True)
    r_hp = jnp.sum(hp, axis=0, keepdims=True)
    r_wh = jnp.sum(wrow * hp, axis=0, keepdims=True)
    r_rg = jnp.sum(relu_g, axis=0, keepdims=True)
    r_lab = jnp.concatenate(
        [jnp.sum(lab, axis=0, keepdims=True), jnp.zeros((1, 112), jnp.float32)],
        axis=1)
    lane128 = lax.broadcasted_iota(jnp.int32, (1, 128), 1)
    r_log = jnp.where(lane128 == 0, jnp.sum(logterm), 0.0)
    add = jnp.concatenate(
        [r_zv, r_hp, r_wh, r_rg, r_lab, r_log, jnp.zeros((2, 128), jnp.float32)],
        axis=0)

    @pl.when(i == 0)
    def _():
        sums_ref[...] = jnp.zeros((8, 128), jnp.float32)

    sums_ref[...] += add

    @pl.when(i == GRID - 1)
    def _():
        _final_combine(sums_ref, lab_any_ref, wp2_ref, wg2_ref, wg1p_ref,
                       bfs_ref, outm_ref, labs_v, labd_v, dsem)


def _final_combine(sums_ref, lab_any_ref, wp2_ref, wg2_ref, wg1p_ref,
                   bfs_ref, out_ref, labs_v, labd_v, dsem):
    s = bfs_ref[0]
    dd = bfs_ref[1]
    cp_s = pltpu.make_async_copy(lab_any_ref.at[pl.ds(s, 1)], labs_v, dsem)
    cp_s.start()
    cp_d = pltpu.make_async_copy(lab_any_ref.at[pl.ds(dd, 1)], labd_v, dsem)
    cp_d.start()
    cp_s.wait()
    cp_d.wait()
    lab_s = labs_v[...]
    lab_d = labd_v[...]
    wg1p = wg1p_ref[...]
    g_s = jnp.dot(lab_s, wg1p, preferred_element_type=jnp.float32)
    g_d = jnp.dot(lab_d, wg1p, preferred_element_type=jnp.float32)
    g_sd = jnp.dot(lab_s + lab_d, wg1p, preferred_element_type=jnp.float32)

    lane16 = lax.broadcasted_iota(jnp.int32, (1, 16), 1)
    stop16 = jnp.where(lane16 == 10, 1.0, 0.0)
    s2 = sums_ref[3:4, :] + jnp.maximum(
        jnp.dot(stop16, wg1p, preferred_element_type=jnp.float32), 0.0)
    n2 = 10001.0
    mean_h2 = (s2 - jnp.maximum(g_d, 0.0) + jnp.maximum(g_sd, 0.0)) / n2
    h2s = jnp.where(s == dd, jnp.maximum(g_sd, 0.0), jnp.maximum(g_s, 0.0))
    ht_head = jnp.dot(mean_h2 + h2s / n2, wg2_ref[...],
                      preferred_element_type=jnp.float32)

    zpocket = jnp.dot((sums_ref[2:3, :] + sums_ref[1:2, :]) / float(N),
                      wp2_ref[...], preferred_element_type=jnp.float32)
    hinit_head = sums_ref[0:1, :] / float(N)
    sumlab = sums_ref[4:5, :]
    hinit_tail = sumlab / float(N)
    lane128 = lax.broadcasted_iota(jnp.int32, (1, 128), 1)
    ht_tail = (sumlab + jnp.where(lane128 == 10, 1.0, 0.0)) / n2
    logrow = sums_ref[5:6, :]
    out_ref[...] = jnp.concatenate(
        [logrow, hinit_head, hinit_tail, ht_head, ht_tail, zpocket,
         jnp.zeros((2, 128), jnp.float32)], axis=0)


def _comb(src, dst, epad):
    e = src.shape[0]
    s = jnp.concatenate([src, jnp.zeros((epad - e,), jnp.int32)])
    d = jnp.concatenate([dst, jnp.full((epad - e,), N, jnp.int32)])
    return jnp.stack([s.reshape(-1, CH), d.reshape(-1, CH)],
                     axis=1).reshape(-1, CH)


def kernel(x_p, edge_index_p, x_l, edge_index_l, bfs_index,
           Wp1, Wp2, Wl1, Wl2, Wg1, Wg2, Wf, bf):
    f32 = jnp.float32
    combp = _comb(edge_index_p[0], edge_index_p[1], EPP)
    combl = _comb(edge_index_l[0], edge_index_l[1], ELP)

    xp_pad = jnp.pad(x_p, ((0, NPAD - N), (0, 0)))
    xpb = xp_pad.astype(jnp.bfloat16)
    xl16 = jnp.pad(
        jnp.concatenate([x_l, jnp.ones((N, 1), f32)], axis=1),
        ((0, NPAD - N), (0, 0)))
    lab16 = jnp.pad(x_l[:, 4:], ((0, NPAD - N), (0, 5)))
    zrowb = jnp.zeros((ROWS_PER_TILE, HID), jnp.bfloat16)
    zrow16 = jnp.zeros((ROWS_PER_TILE, 16), f32)
    ones16 = jnp.ones((CH, 16), f32)

    aggp2, degp2, aggl2 = _sc_stage_a(
        xpb, xl16, combp, combl, zrowb, zrow16, ones16)
    aggp_f = [aggp2[0], aggp2[1]]
    degp = [degp2[0], degp2[1]]

    wl1p = jnp.pad(Wl1, ((0, 1), (0, 0)))
    row_spec = pl.BlockSpec((BR, HID), lambda i: (i, 0))
    row16_spec = pl.BlockSpec((BR, 16), lambda i: (i, 0))
    w_spec = pl.BlockSpec((HID, HID), lambda i: (0, 0))
    hp, z1l, invp16, invl16 = pl.pallas_call(
        _tc_stage_b,
        grid=(GRID,),
        in_specs=[row_spec, row_spec, row_spec, row16_spec, row16_spec,
                  row16_spec, row16_spec, row16_spec, w_spec,
                  pl.BlockSpec((16, HID), lambda i: (0, 0))],
        out_specs=[row_spec, row_spec, row16_spec, row16_spec],
        out_shape=[
            jax.ShapeDtypeStruct((NPAD, HID), f32),
            jax.ShapeDtypeStruct((NPAD, HID), jnp.bfloat16),
            jax.ShapeDtypeStruct((NPAD, 16), f32),
            jax.ShapeDtypeStruct((NPAD, 16), f32),
        ],
    )(xp_pad, aggp_f[0], aggp_f[1], degp[0], degp[1],
      xl16, aggl2[0], aggl2[1], Wp1, wl1p)

    aggl2p, w2 = _sc_stage_c(
        z1l, invp16, combp, combl, zrowb, zrow16)

    wf16 = jnp.pad(Wf, ((0, 0), (0, 5)))
    bf16 = jnp.pad(bf, (0, 5)).reshape(1, 16)
    wg1p = jnp.pad(Wg1, ((0, 5), (0, 0)))
    _, outm = pl.pallas_call(
        _tc_stage_d,
        grid=(GRID,),
        in_specs=[row_spec, row_spec, row_spec, row_spec, row16_spec,
                  row16_spec, row16_spec, row16_spec, w_spec,
                  pl.BlockSpec((HID, 16), lambda i: (0, 0)),
                  pl.BlockSpec((1, 16), lambda i: (0, 0)),
                  pl.BlockSpec((16, HID), lambda i: (0, 0)),
                  w_spec, w_spec,
                  pl.BlockSpec(memory_space=pl.ANY),
                  pl.BlockSpec(memory_space=pltpu.SMEM)],
        out_specs=[pl.BlockSpec((8, 128), lambda i: (0, 0)),
                   pl.BlockSpec((8, 128), lambda i: (0, 0))],
        out_shape=[jax.ShapeDtypeStruct((8, 128), f32),
                   jax.ShapeDtypeStruct((8, 128), f32)],
        scratch_shapes=[pltpu.VMEM((1, 16), f32), pltpu.VMEM((1, 16), f32),
                        pltpu.SemaphoreType.DMA],
    )(hp, z1l, aggl2p[0], aggl2p[1], invl16, lab16, w2[0], w2[1],
      Wl2, wf16, bf16, wg1p, Wp2, Wg2, lab16, bfs_index[0])

    return jnp.concatenate([outm[0, 0:1], outm[1], outm[2, :11], outm[3],
                            outm[4, :11], outm[5]])
